# baseline (device time: 358784 ns/iter reference)
import jax
import jax.numpy as jnp
from jax import lax
from jax.experimental import pallas as pl
from jax.experimental.pallas import tpu as pltpu

N_DEV = 32
NSUB = 4
NSLOT = 3

HOPS = {(0, 0): 16, (0, 1): 16, (0, 2): 15, (0, 3): 15,
        (1, 0): 15, (1, 1): 15, (1, 2): 16, (1, 3): 16}
HMAX = 16


def _perm(p):
    x = p // 16
    q = jnp.where(x == 0, p, 31 - p)
    yy = q // 4
    zr = q % 4
    zz = jnp.where(yy % 2 == 0, zr, 3 - zr)
    pidx = 2 * yy + jnp.where(x == 0, yy % 2, 1 - yy % 2)
    return 8 * zz + pidx


def _inv(lid):
    zz = lid // 8
    pi = lid % 8
    yy = pi // 2
    x = ((pi + 1) // 2) % 2
    q = 4 * yy + jnp.where(yy % 2 == 0, zz, 3 - zz)
    return jnp.where(x == 0, q, 31 - q)


def _gelu(y):
    c = 0.7978845608028654
    return 0.5 * y * (1.0 + jnp.tanh(c * (y + 0.044715 * y * y * y)))


def kernel(x, w_mat):
    m_per, k = x.shape
    _, n_per = w_mat.shape
    m_sub = m_per // NSUB

    def body(x_ref, w_ref, out_ref, comm_ref, send_sems, recv_sems,
             credit_sems):
        my = lax.axis_index("i")
        pos = _inv(my)
        right = _perm(lax.rem(pos + 1, N_DEV))
        left = _perm(lax.rem(pos + N_DEV - 1, N_DEV))

        credit = {(d, c): credit_sems.at[d, c]
                  for d in (0, 1) for c in range(NSUB)}
        dst_of = {0: right, 1: left}
        upstream_of = {0: left, 1: right}

        barrier_sem = pltpu.get_barrier_semaphore()
        for nbr in (left, right):
            pl.semaphore_signal(
                barrier_sem, inc=1,
                device_id=(nbr,), device_id_type=pl.DeviceIdType.MESH,
            )
        pl.semaphore_wait(barrier_sem, 2)

        def origin(d, h):
            if d == 0:
                return _perm(lax.rem(pos - h - 1 + N_DEV, N_DEV))
            return _perm(lax.rem(pos + h + 1, N_DEV))

        stream_order = [(0, 0), (1, 2), (0, 1), (1, 3),
                        (0, 2), (1, 0), (0, 3), (1, 1)]
        prev = {}
        for h in range(HMAX + 1):
            for d, c in stream_order:
                H = HOPS[(d, c)]
                if 1 <= h <= H:
                    prev[(d, c)].wait()
                    if h - 1 <= H - NSLOT:
                        pl.semaphore_signal(
                            credit[(d, c)], inc=1,
                            device_id=(upstream_of[d],),
                            device_id_type=pl.DeviceIdType.MESH,
                        )
                if h < H:
                    if h >= NSLOT - 1:
                        pl.semaphore_wait(credit[(d, c)], 1)
                    src = (
                        x_ref.at[pl.ds(c * m_sub, m_sub), :]
                        if h == 0
                        else comm_ref.at[d, c, h % NSLOT]
                    )
                    rdma = pltpu.make_async_remote_copy(
                        src_ref=src,
                        dst_ref=comm_ref.at[d, c, (h + 1) % NSLOT],
                        send_sem=send_sems.at[d, c, h % NSLOT],
                        recv_sem=recv_sems.at[d, c, (h + 1) % NSLOT],
                        device_id=(dst_of[d],),
                        device_id_type=pl.DeviceIdType.MESH,
                    )
                    rdma.start()
                    prev[(d, c)] = rdma

            if h == 0:
                y = jnp.dot(
                    x_ref[...], w_ref[...], preferred_element_type=jnp.float32
                )
                out_ref[pl.ds(my * m_per, m_per), :] = _gelu(y)
            if h >= 1:
                for d, c in stream_order:
                    if h - 1 < HOPS[(d, c)]:
                        og = origin(d, h - 1)
                        y = jnp.dot(
                            comm_ref[d, c, h % NSLOT], w_ref[...],
                            preferred_element_type=jnp.float32,
                        )
                        out_ref[
                            pl.ds(og * m_per + c * m_sub, m_sub), :
                        ] = _gelu(y)

    out_shape = jax.ShapeDtypeStruct((N_DEV * m_per, n_per), jnp.float32)
    return pl.pallas_call(
        body,
        out_shape=out_shape,
        in_specs=[
            pl.BlockSpec(memory_space=pltpu.VMEM),
            pl.BlockSpec(memory_space=pltpu.VMEM),
        ],
        out_specs=pl.BlockSpec(memory_space=pltpu.VMEM),
        scratch_shapes=[
            pltpu.VMEM((2, NSUB, NSLOT, m_per // NSUB, k), jnp.float32),
            pltpu.SemaphoreType.DMA((2, NSUB, NSLOT)),
            pltpu.SemaphoreType.DMA((2, NSUB, NSLOT)),
            pltpu.SemaphoreType.REGULAR((2, NSUB)),
        ],
        compiler_params=pltpu.CompilerParams(collective_id=0),
    )(x, w_mat)


# device time: 357947 ns/iter; 1.0023x vs baseline; 1.0023x over previous
import jax
import jax.numpy as jnp
from jax import lax
from jax.experimental import pallas as pl
from jax.experimental.pallas import tpu as pltpu

N_DEV = 32
NSUB = 2
NSLOT = 3

HOPS = {(0, 0): 16, (0, 1): 15, (1, 0): 15, (1, 1): 16}
HMAX = 16


def _perm(p):
    x = p // 16
    q = jnp.where(x == 0, p, 31 - p)
    yy = q // 4
    zr = q % 4
    zz = jnp.where(yy % 2 == 0, zr, 3 - zr)
    pidx = 2 * yy + jnp.where(x == 0, yy % 2, 1 - yy % 2)
    return 8 * zz + pidx


def _inv(lid):
    zz = lid // 8
    pi = lid % 8
    yy = pi // 2
    x = ((pi + 1) // 2) % 2
    q = 4 * yy + jnp.where(yy % 2 == 0, zz, 3 - zz)
    return jnp.where(x == 0, q, 31 - q)


def _gelu(y):
    c = 0.7978845608028654
    return 0.5 * y * (1.0 + jnp.tanh(c * (y + 0.044715 * y * y * y)))


def kernel(x, w_mat):
    m_per, k = x.shape
    _, n_per = w_mat.shape
    m_sub = m_per // NSUB

    def body(x_ref, w_ref, out_ref, comm_ref, send_sems, recv_sems,
             cr_r0, cr_r1, cr_l0, cr_l1):
        my = lax.axis_index("i")
        pos = _inv(my)
        right = _perm(lax.rem(pos + 1, N_DEV))
        left = _perm(lax.rem(pos + N_DEV - 1, N_DEV))

        credit = {(0, 0): cr_r0, (0, 1): cr_r1, (1, 0): cr_l0, (1, 1): cr_l1}
        dst_of = {0: right, 1: left}
        upstream_of = {0: left, 1: right}

        barrier_sem = pltpu.get_barrier_semaphore()
        for nbr in (left, right):
            pl.semaphore_signal(
                barrier_sem, inc=1,
                device_id=(nbr,), device_id_type=pl.DeviceIdType.MESH,
            )
        pl.semaphore_wait(barrier_sem, 2)

        def origin(d, h):
            if d == 0:
                return _perm(lax.rem(pos - h - 1 + N_DEV, N_DEV))
            return _perm(lax.rem(pos + h + 1, N_DEV))

        stream_order = [(0, 0), (1, 1), (0, 1), (1, 0)]
        prev = {}
        for h in range(HMAX + 1):
            for d, c in stream_order:
                H = HOPS[(d, c)]
                if 1 <= h <= H:
                    prev[(d, c)].wait()
                    if h - 1 <= H - NSLOT:
                        pl.semaphore_signal(
                            credit[(d, c)], inc=1,
                            device_id=(upstream_of[d],),
                            device_id_type=pl.DeviceIdType.MESH,
                        )
                if h < H:
                    if h >= NSLOT - 1:
                        pl.semaphore_wait(credit[(d, c)], 1)
                    src = (
                        x_ref.at[pl.ds(c * m_sub, m_sub), :]
                        if h == 0
                        else comm_ref.at[d, c, h % NSLOT]
                    )
                    rdma = pltpu.make_async_remote_copy(
                        src_ref=src,
                        dst_ref=comm_ref.at[d, c, (h + 1) % NSLOT],
                        send_sem=send_sems.at[d, c, h % NSLOT],
                        recv_sem=recv_sems.at[d, c, (h + 1) % NSLOT],
                        device_id=(dst_of[d],),
                        device_id_type=pl.DeviceIdType.MESH,
                    )
                    rdma.start()
                    prev[(d, c)] = rdma

            if h == 0:
                y = jnp.dot(
                    x_ref[...], w_ref[...], preferred_element_type=jnp.float32
                )
                out_ref[pl.ds(my * m_per, m_per), :] = _gelu(y)
            if h >= 1:
                for d, c in stream_order:
                    if h - 1 < HOPS[(d, c)]:
                        og = origin(d, h - 1)
                        y = jnp.dot(
                            comm_ref[d, c, h % NSLOT], w_ref[...],
                            preferred_element_type=jnp.float32,
                        )
                        out_ref[
                            pl.ds(og * m_per + c * m_sub, m_sub), :
                        ] = _gelu(y)

    out_shape = jax.ShapeDtypeStruct((N_DEV * m_per, n_per), jnp.float32)
    return pl.pallas_call(
        body,
        out_shape=out_shape,
        in_specs=[
            pl.BlockSpec(memory_space=pltpu.VMEM),
            pl.BlockSpec(memory_space=pltpu.VMEM),
        ],
        out_specs=pl.BlockSpec(memory_space=pltpu.VMEM),
        scratch_shapes=[
            pltpu.VMEM((2, NSUB, NSLOT, m_per // NSUB, k), jnp.float32),
            pltpu.SemaphoreType.DMA((2, NSUB, NSLOT)),
            pltpu.SemaphoreType.DMA((2, NSUB, NSLOT)),
            pltpu.SemaphoreType.REGULAR,
            pltpu.SemaphoreType.REGULAR,
            pltpu.SemaphoreType.REGULAR,
            pltpu.SemaphoreType.REGULAR,
        ],
        compiler_params=pltpu.CompilerParams(collective_id=0),
    )(x, w_mat)
